# Initial kernel scaffold; baseline (speedup 1.0000x reference)
import functools

import jax
import jax.numpy as jnp
from jax import lax
from jax.experimental import pallas as pl
from jax.experimental.pallas import tpu as pltpu
from jax.experimental.pallas import tpu_sc as plsc

D_MODEL = 64
NUM_WORKERS = 32
CHUNK = 512


def _gather_body(nsteps, loc_hbm, table_hbm, out_hbm, idx_v, rows_v, sem):
    nc = 2
    wid = lax.axis_index("s") * nc + lax.axis_index("c")
    per_w = nsteps * CHUNK
    base = wid * per_w

    def step(i, _):
        off = base + i * CHUNK
        pltpu.sync_copy(loc_hbm.at[pl.ds(off, CHUNK)], idx_v)
        pltpu.async_copy(table_hbm.at[idx_v], rows_v, sem).wait()
        pltpu.sync_copy(rows_v, out_hbm.at[pl.ds(off, CHUNK)])
        return 0

    lax.fori_loop(0, nsteps, step, 0)


def kernel(locations, table):
    b, t = locations.shape
    n = b * t
    assert n % (NUM_WORKERS * CHUNK) == 0
    nsteps = n // (NUM_WORKERS * CHUNK)
    flat = locations.reshape(n).astype(jnp.int32)

    mesh = plsc.VectorSubcoreMesh(core_axis_name="c", subcore_axis_name="s")
    run = pl.kernel(
        functools.partial(_gather_body, nsteps),
        mesh=mesh,
        out_type=jax.ShapeDtypeStruct((n, D_MODEL), jnp.float32),
        scratch_types=[
            pltpu.VMEM((CHUNK,), jnp.int32),
            pltpu.VMEM((CHUNK, D_MODEL), jnp.float32),
            pltpu.SemaphoreType.DMA,
        ],
    )
    out = run(flat, table)
    return out.reshape(b, t, D_MODEL)


# speedup vs baseline: 1.7946x; 1.7946x over previous
import functools

import jax
import jax.numpy as jnp
from jax import lax
from jax.experimental import pallas as pl
from jax.experimental.pallas import tpu as pltpu
from jax.experimental.pallas import tpu_sc as plsc

D_MODEL = 64
NUM_WORKERS = 32
CHUNK = 512


def _gather_body(nsteps, loc_hbm, table_hbm, out_hbm, idx_v, rows_v, sem):
    nc = 2
    wid = lax.axis_index("s") * nc + lax.axis_index("c")
    per_w = nsteps * CHUNK
    base = wid * per_w

    def step(i, _):
        off = base + i * CHUNK
        pltpu.sync_copy(loc_hbm.at[pl.ds(off, CHUNK)], idx_v)
        pltpu.async_copy(table_hbm.at[idx_v], rows_v, sem).wait()
        pltpu.sync_copy(rows_v, out_hbm.at[pl.ds(off, CHUNK)])
        return 0

    lax.fori_loop(0, nsteps, step, 0)


def kernel(locations, table):
    b, t = locations.shape
    n = b * t
    assert n % (NUM_WORKERS * CHUNK) == 0
    nsteps = n // (NUM_WORKERS * CHUNK)
    flat = locations.reshape(n).astype(jnp.int32)

    mesh = plsc.VectorSubcoreMesh(core_axis_name="c", subcore_axis_name="s")
    run = pl.kernel(
        functools.partial(_gather_body, nsteps),
        mesh=mesh,
        out_type=jax.ShapeDtypeStruct((n, D_MODEL), jnp.float32),
        scratch_types=[
            pltpu.VMEM((CHUNK,), jnp.int32),
            pltpu.VMEM((CHUNK, D_MODEL), jnp.float32),
            pltpu.SemaphoreType.DMA,
        ],
        compiler_params=pltpu.CompilerParams(use_tc_tiling_on_sc=False),
    )
    out = run(flat, table)
    return out.reshape(b, t, D_MODEL)


# speedup vs baseline: 1.8728x; 1.0436x over previous
import functools

import jax
import jax.numpy as jnp
from jax import lax
from jax.experimental import pallas as pl
from jax.experimental.pallas import tpu as pltpu
from jax.experimental.pallas import tpu_sc as plsc

D_MODEL = 64
NUM_WORKERS = 32
CHUNK = 512


def _gather_body(nsteps, loc_hbm, table_hbm, out_hbm,
                 idx_v, rows0, rows1, gs0, gs1, ss0, ss1):
    nc = 2
    wid = lax.axis_index("s") * nc + lax.axis_index("c")
    base = wid * nsteps * CHUNK
    pltpu.sync_copy(loc_hbm.at[pl.ds(wid * nsteps, nsteps)], idx_v)

    rows = (rows0, rows1)
    gs = (gs0, gs1)
    ss = (ss0, ss1)

    def fire_gather(i, b):
        pltpu.async_copy(table_hbm.at[idx_v.at[i]], rows[b], gs[b])

    def wait_gather(b):
        pltpu.make_async_copy(table_hbm.at[idx_v.at[0]],
                              rows[b], gs[b]).wait()

    def fire_store(i, b):
        pltpu.async_copy(rows[b], out_hbm.at[pl.ds(base + i * CHUNK, CHUNK)],
                         ss[b])

    def wait_store(b):
        pltpu.make_async_copy(rows[b], out_hbm.at[pl.ds(base, CHUNK)],
                              ss[b]).wait()

    fire_gather(0, 0)
    nit = nsteps // 2

    def it_body(it, _):
        i0 = 2 * it

        @pl.when(it > 0)
        def _():
            wait_store(1)
        fire_gather(i0 + 1, 1)
        wait_gather(0)
        fire_store(i0, 0)

        wait_store(0)

        @pl.when(it < nit - 1)
        def _():
            fire_gather(i0 + 2, 0)
        wait_gather(1)
        fire_store(i0 + 1, 1)
        return 0

    lax.fori_loop(0, nit, it_body, 0)
    wait_store(1)


def kernel(locations, table):
    b, t = locations.shape
    n = b * t
    assert n % (NUM_WORKERS * CHUNK) == 0
    nsteps = n // (NUM_WORKERS * CHUNK)
    assert nsteps % 2 == 0
    flat = locations.reshape(n // CHUNK, CHUNK).astype(jnp.int32)

    mesh = plsc.VectorSubcoreMesh(core_axis_name="c", subcore_axis_name="s")
    run = pl.kernel(
        functools.partial(_gather_body, nsteps),
        mesh=mesh,
        out_type=jax.ShapeDtypeStruct((n, D_MODEL), jnp.float32),
        scratch_types=[
            pltpu.VMEM((nsteps, CHUNK), jnp.int32),
            pltpu.VMEM((CHUNK, D_MODEL), jnp.float32),
            pltpu.VMEM((CHUNK, D_MODEL), jnp.float32),
            pltpu.SemaphoreType.DMA,
            pltpu.SemaphoreType.DMA,
            pltpu.SemaphoreType.DMA,
            pltpu.SemaphoreType.DMA,
        ],
        compiler_params=pltpu.CompilerParams(use_tc_tiling_on_sc=False),
    )
    out = run(flat, table)
    return out.reshape(b, t, D_MODEL)
